# Initial kernel scaffold; baseline (speedup 1.0000x reference)
import jax
import jax.numpy as jnp
from jax.experimental import pallas as pl


def kernel(spd_index, spd_lengths, batch, edge_index, e2e_spd_index, e2e_spd_lengths, e_batch, e2e_edge_index, W_spd, W_e2e):
    raise NotImplementedError("write your pallas kernel here")


# speedup vs baseline: 65.6565x; 65.6565x over previous
import functools

import jax
import jax.numpy as jnp
from jax import lax
from jax.experimental import pallas as pl
from jax.experimental.pallas import tpu as pltpu
from jax.experimental.pallas import tpu_sc as plsc

NW = 32
NC = 2
L = 16
B = 128
G = B // NW

N1, PPG1, EPG1 = 32, 256, 64
N2, PPG2, EPG2 = 64, 512, 128
E1 = B * EPG1
E2 = B * EPG2


def _half(N, ppg, epg, psrc_h, pdst_h, plen_h, esrc_h, edst_h, out_h,
          psrc, pdst, plen, esrc, edst, slot_e, smap, q, qv, wid):
    npr = G * ppg
    ned = G * epg
    nself = G * N
    cells = G * N * N
    pbase = wid * npr
    ebase = wid * ned
    cell_off = wid * cells
    iota = lax.iota(jnp.int32, L)
    ones = jnp.ones((L,), jnp.float32)

    pltpu.sync_copy(psrc_h.at[pl.ds(pbase, npr)], psrc)
    pltpu.sync_copy(pdst_h.at[pl.ds(pbase, npr)], pdst)
    pltpu.sync_copy(plen_h.at[pl.ds(pbase, npr)], plen)
    pltpu.sync_copy(esrc_h.at[pl.ds(ebase, ned)], esrc)
    pltpu.sync_copy(edst_h.at[pl.ds(ebase, ned)], edst)

    dummy = jnp.full((L,), ned, jnp.int32)

    def init_s(i, c):
        smap[pl.ds(i * L, L)] = dummy
        return c

    lax.fori_loop(0, cells // L, init_s, 0)

    zf = jnp.zeros((L,), jnp.float32)

    def init_q(i, c):
        q[pl.ds(i * L, L)] = zf
        return c

    lax.fori_loop(0, ned + 1, init_q, 0)

    mask_n = N - 1

    def ekey(i):
        s = esrc[pl.ds(i * L, L)]
        d = edst[pl.ds(i * L, L)]
        return s * N + (d & mask_n) - cell_off

    def scat_e(i, c):
        plsc.store_scatter(smap, [ekey(i)], i * L + iota)
        return c

    lax.fori_loop(0, ned // L, scat_e, 0)

    def gath_e(i, c):
        slot_e[pl.ds(i * L, L)] = plsc.load_gather(smap, [ekey(i)])
        return c

    lax.fori_loop(0, ned // L, gath_e, 0)

    def pair_step(i, c):
        s = psrc[pl.ds(i * L, L)]
        d = pdst[pl.ds(i * L, L)]
        t = plen[pl.ds(i * L, L)] + 1
        k = s * N + (d & mask_n) - cell_off
        slot = plsc.load_gather(smap, [k])
        plsc.addupdate_scatter(q, [slot * L + t], ones)
        return c

    lax.fori_loop(0, npr // L, pair_step, 0)

    def self_step(i, c):
        iloc = i * L + iota
        k = iloc * N + (iloc & mask_n)
        slot = plsc.load_gather(smap, [k])
        plsc.addupdate_scatter(q, [slot * L], ones)
        return c

    lax.fori_loop(0, nself // L, self_step, 0)

    def out_step(i, c):
        slot = slot_e[pl.ds(i * L, L)] * L
        dbase = (i * L + iota) * L
        for t in range(L):
            vals = plsc.load_gather(q, [slot + t])
            plsc.store_scatter(qv, [dbase + t], vals)
        return c

    lax.fori_loop(0, ned // L, out_step, 0)

    pltpu.sync_copy(qv, out_h.at[pl.ds(ebase * L, ned * L)])


def _sc_body(psrc1, pdst1, plen1, esrc1, edst1,
             psrc2, pdst2, plen2, esrc2, edst2,
             q1_out, q2_out,
             psrc1_v, pdst1_v, plen1_v, esrc1_v, edst1_v, slot1_v,
             smap1_v, q1_v, qv1_v,
             psrc2_v, pdst2_v, plen2_v, esrc2_v, edst2_v, slot2_v,
             smap2_v, q2_v, qv2_v):
    wid = lax.axis_index("s") * NC + lax.axis_index("c")
    _half(N1, PPG1, EPG1, psrc1, pdst1, plen1, esrc1, edst1, q1_out,
          psrc1_v, pdst1_v, plen1_v, esrc1_v, edst1_v, slot1_v,
          smap1_v, q1_v, qv1_v, wid)
    _half(N2, PPG2, EPG2, psrc2, pdst2, plen2, esrc2, edst2, q2_out,
          psrc2_v, pdst2_v, plen2_v, esrc2_v, edst2_v, slot2_v,
          smap2_v, q2_v, qv2_v, wid)


def _half_scratch(N, ppg, epg):
    npr = G * ppg
    ned = G * epg
    cells = G * N * N
    return [
        pltpu.VMEM((npr,), jnp.int32),
        pltpu.VMEM((npr,), jnp.int32),
        pltpu.VMEM((npr,), jnp.int32),
        pltpu.VMEM((ned,), jnp.int32),
        pltpu.VMEM((ned,), jnp.int32),
        pltpu.VMEM((ned,), jnp.int32),
        pltpu.VMEM((cells,), jnp.int32),
        pltpu.VMEM(((ned + 1) * L,), jnp.float32),
        pltpu.VMEM((ned * L,), jnp.float32),
    ]


_sc_counts = pl.kernel(
    _sc_body,
    out_type=(jax.ShapeDtypeStruct((E1 * L,), jnp.float32),
              jax.ShapeDtypeStruct((E2 * L,), jnp.float32)),
    mesh=plsc.VectorSubcoreMesh(core_axis_name="c", subcore_axis_name="s"),
    scratch_types=_half_scratch(N1, PPG1, EPG1) + _half_scratch(N2, PPG2, EPG2),
    compiler_params=pltpu.CompilerParams(needs_layout_passes=False),
)


def _tc_body(q1_ref, w1_ref, q2_ref, w2_ref, o1_ref, o2_ref):
    o1_ref[...] = jnp.dot(q1_ref[...], w1_ref[...],
                          preferred_element_type=jnp.float32)
    o2_ref[...] = jnp.dot(q2_ref[...], w2_ref[...],
                          preferred_element_type=jnp.float32)


_tc_project = pl.pallas_call(
    _tc_body,
    out_shape=(jax.ShapeDtypeStruct((E1, 64), jnp.float32),
               jax.ShapeDtypeStruct((E2, 64), jnp.float32)),
)


def kernel(spd_index, spd_lengths, batch, edge_index,
           e2e_spd_index, e2e_spd_lengths, e_batch, e2e_edge_index,
           W_spd, W_e2e):
    del batch, e_batch
    q1, q2 = _sc_counts(
        spd_index[0], spd_index[1], spd_lengths,
        edge_index[0], edge_index[1],
        e2e_spd_index[0], e2e_spd_index[1], e2e_spd_lengths,
        e2e_edge_index[0], e2e_edge_index[1],
    )
    w1 = jnp.zeros((L, 64), jnp.float32).at[:12].set(W_spd)
    w2 = jnp.zeros((L, 64), jnp.float32).at[:12].set(W_e2e)
    return _tc_project(q1.reshape(E1, L), w1, q2.reshape(E2, L), w2)


# speedup vs baseline: 96.6952x; 1.4727x over previous
import jax
import jax.numpy as jnp
from jax import lax
from jax.experimental import pallas as pl
from jax.experimental.pallas import tpu as pltpu
from jax.experimental.pallas import tpu_sc as plsc

NW = 32
NC = 2
L = 16
B = 128
G = B // NW
QSTR = 17

N1, PPG1, EPG1 = 32, 256, 64
N2, PPG2, EPG2 = 64, 512, 128
E1 = B * EPG1
E2 = B * EPG2


def _qwords(epg):
    return ((G * epg + 1) * QSTR + L - 1) // L * L


def _memset(ref, nvec, vec):
    def body(i, c):
        base = i * (4 * L)
        for j in range(4):
            ref[pl.ds(base + j * L, L)] = vec
        return c
    lax.fori_loop(0, nvec // 4, body, 0)
    for j in range(nvec // 4 * 4, nvec):
        ref[pl.ds(j * L, L)] = vec


def _half_compute(N, ppg, epg, out_h,
                  psrc, pdst, plen, esrc, edst, smap, q, qv, wid):
    npr = G * ppg
    ned = G * epg
    nself = G * N
    cells = G * N * N
    ebase = wid * ned
    cell_off = wid * cells
    iota = lax.iota(jnp.int32, L)
    ones = jnp.ones((L,), jnp.float32)
    mask_n = N - 1

    def ekey(i):
        s = esrc[pl.ds(i * L, L)]
        d = edst[pl.ds(i * L, L)]
        return s * N + (d & mask_n) - cell_off

    def scat_e(i, c):
        plsc.store_scatter(smap, [ekey(i)], i * L + iota)
        return c

    lax.fori_loop(0, ned // L, scat_e, 0)

    def pair_step(i, c):
        s = psrc[pl.ds(i * L, L)]
        d = pdst[pl.ds(i * L, L)]
        t = plen[pl.ds(i * L, L)] + 1
        k = s * N + (d & mask_n) - cell_off
        slot = plsc.load_gather(smap, [k])
        plsc.addupdate_scatter(q, [slot * QSTR + t], ones)
        return c

    lax.fori_loop(0, npr // L, pair_step, 0)

    def self_step(i, c):
        iloc = i * L + iota
        k = iloc * N + (iloc & mask_n)
        slot = plsc.load_gather(smap, [k])
        plsc.addupdate_scatter(q, [slot * QSTR], ones)
        return c

    lax.fori_loop(0, nself // L, self_step, 0)

    def out_step(i, c):
        slot = plsc.load_gather(smap, [ekey(i)]) * QSTR
        for t in range(L):
            vals = plsc.load_gather(q, [slot + t])
            qv[t, pl.ds(i * L, L)] = vals
        return c

    lax.fori_loop(0, ned // L, out_step, 0)

    pltpu.sync_copy(qv, out_h.at[:, pl.ds(ebase, ned)])


def _sc_body(psrc1_h, pdst1_h, plen1_h, esrc1_h, edst1_h,
             psrc2_h, pdst2_h, plen2_h, esrc2_h, edst2_h,
             q1_out, q2_out,
             psrc1, pdst1, plen1, esrc1, edst1, smap1, q1, qv1,
             psrc2, pdst2, plen2, esrc2, edst2, smap2, q2, qv2,
             sem):
    wid = lax.axis_index("s") * NC + lax.axis_index("c")
    npr1, ned1 = G * PPG1, G * EPG1
    npr2, ned2 = G * PPG2, G * EPG2

    cps = []
    for hbm, vmem, base, n in (
            (psrc1_h, psrc1, wid * npr1, npr1),
            (pdst1_h, pdst1, wid * npr1, npr1),
            (plen1_h, plen1, wid * npr1, npr1),
            (esrc1_h, esrc1, wid * ned1, ned1),
            (edst1_h, edst1, wid * ned1, ned1),
            (psrc2_h, psrc2, wid * npr2, npr2),
            (pdst2_h, pdst2, wid * npr2, npr2),
            (plen2_h, plen2, wid * npr2, npr2),
            (esrc2_h, esrc2, wid * ned2, ned2),
            (edst2_h, edst2, wid * ned2, ned2)):
        cps.append(pltpu.async_copy(hbm.at[pl.ds(base, n)], vmem, sem))

    _memset(smap1, G * N1 * N1 // L, jnp.full((L,), ned1, jnp.int32))
    _memset(smap2, G * N2 * N2 // L, jnp.full((L,), ned2, jnp.int32))
    zf = jnp.zeros((L,), jnp.float32)
    _memset(q1, _qwords(EPG1) // L, zf)
    _memset(q2, _qwords(EPG2) // L, zf)

    for cp in cps:
        cp.wait()

    _half_compute(N1, PPG1, EPG1, q1_out,
                  psrc1, pdst1, plen1, esrc1, edst1, smap1, q1, qv1, wid)
    _half_compute(N2, PPG2, EPG2, q2_out,
                  psrc2, pdst2, plen2, esrc2, edst2, smap2, q2, qv2, wid)


def _half_scratch(N, ppg, epg):
    npr = G * ppg
    ned = G * epg
    cells = G * N * N
    return [
        pltpu.VMEM((npr,), jnp.int32),
        pltpu.VMEM((npr,), jnp.int32),
        pltpu.VMEM((npr,), jnp.int32),
        pltpu.VMEM((ned,), jnp.int32),
        pltpu.VMEM((ned,), jnp.int32),
        pltpu.VMEM((cells,), jnp.int32),
        pltpu.VMEM((_qwords(epg),), jnp.float32),
        pltpu.VMEM((L, ned), jnp.float32),
    ]


_sc_counts = pl.kernel(
    _sc_body,
    out_type=(jax.ShapeDtypeStruct((L, E1), jnp.float32),
              jax.ShapeDtypeStruct((L, E2), jnp.float32)),
    mesh=plsc.VectorSubcoreMesh(core_axis_name="c", subcore_axis_name="s"),
    scratch_types=_half_scratch(N1, PPG1, EPG1) + _half_scratch(N2, PPG2, EPG2)
    + [pltpu.SemaphoreType.DMA],
    compiler_params=pltpu.CompilerParams(needs_layout_passes=False),
)


def _tc_body(q1_ref, w1_ref, q2_ref, w2_ref, o1_ref, o2_ref):
    dn = (((0,), (0,)), ((), ()))
    o1_ref[...] = lax.dot_general(q1_ref[...], w1_ref[...], dn,
                                  preferred_element_type=jnp.float32)
    o2_ref[...] = lax.dot_general(q2_ref[...], w2_ref[...], dn,
                                  preferred_element_type=jnp.float32)


_tc_project = pl.pallas_call(
    _tc_body,
    out_shape=(jax.ShapeDtypeStruct((E1, 64), jnp.float32),
               jax.ShapeDtypeStruct((E2, 64), jnp.float32)),
)


def kernel(spd_index, spd_lengths, batch, edge_index,
           e2e_spd_index, e2e_spd_lengths, e_batch, e2e_edge_index,
           W_spd, W_e2e):
    del batch, e_batch
    q1, q2 = _sc_counts(
        spd_index[0], spd_index[1], spd_lengths,
        edge_index[0], edge_index[1],
        e2e_spd_index[0], e2e_spd_index[1], e2e_spd_lengths,
        e2e_edge_index[0], e2e_edge_index[1],
    )
    w1 = jnp.zeros((L, 64), jnp.float32).at[:12].set(W_spd)
    w2 = jnp.zeros((L, 64), jnp.float32).at[:12].set(W_e2e)
    return _tc_project(q1, w1, q2, w2)
